# Initial kernel scaffold; baseline (speedup 1.0000x reference)
import jax
import jax.numpy as jnp
from jax.experimental import pallas as pl


def kernel(x, tok_table, pos_table, gamma, beta):
    raise NotImplementedError("write your pallas kernel here")


# speedup vs baseline: 1.8356x; 1.8356x over previous
import functools

import jax
import jax.numpy as jnp
from jax import lax
from jax.experimental import pallas as pl
from jax.experimental.pallas import tpu as pltpu
from jax.experimental.pallas import tpu_sc as plsc

D = 128
L = 16
ND = D // L
S = 200
CHUNK = 128
EPS = 1e-12


def _rsqrt(x):
    i = lax.bitcast_convert_type(x, jnp.int32)
    i = jnp.int32(0x5F3759DF) - lax.shift_right_logical(i, 1)
    y = lax.bitcast_convert_type(i, jnp.float32)
    for _ in range(3):
        y = y * (1.5 - 0.5 * x * y * y)
    return y


def _tree_sum(vs):
    while len(vs) > 1:
        vs = [a + b for a, b in zip(vs[::2], vs[1::2])]
    return vs[0]


def _make_sc_kernel(total_tokens):
    info = plsc.get_sparse_core_info()
    nc, ns = info.num_cores, info.num_subcores
    nw = nc * ns
    tok_per_w = total_tokens // nw
    chunks_per_w = tok_per_w // CHUNK

    mesh = plsc.VectorSubcoreMesh(core_axis_name="c", subcore_axis_name="s")

    @functools.partial(
        pl.kernel,
        mesh=mesh,
        out_type=jax.ShapeDtypeStruct((total_tokens, D), jnp.float32),
        compiler_params=pltpu.CompilerParams(needs_layout_passes=False),
        scratch_types=[
            pltpu.VMEM((chunks_per_w, CHUNK), jnp.int32),
            pltpu.VMEM((2 * S, D), jnp.float32),
            pltpu.VMEM((CHUNK, D), jnp.float32),
            pltpu.VMEM((2, D), jnp.float32),
            pltpu.SemaphoreType.DMA,
        ],
    )
    def sc_kernel(x_hbm, tok_hbm, pos2_hbm, gb_hbm, out_hbm,
                  idx_v, pos_v, rows_v, gb_v, sem):
        wid = lax.axis_index("s") * nc + lax.axis_index("c")
        base = wid * tok_per_w

        pltpu.sync_copy(x_hbm.at[wid], idx_v)
        pltpu.sync_copy(pos2_hbm, pos_v)
        pltpu.sync_copy(gb_hbm, gb_v)
        g = [gb_v[0, pl.ds(j * L, L)] for j in range(ND)]
        b = [gb_v[1, pl.ds(j * L, L)] for j in range(ND)]

        def chunk_body(c, carry):
            pltpu.async_copy(tok_hbm.at[idx_v.at[c]], rows_v, sem).wait()
            s0 = lax.rem(c * CHUNK, S)

            def tok_body(t, carry2):
                p = s0 + t
                v = [rows_v[t, pl.ds(j * L, L)] + pos_v[p, pl.ds(j * L, L)]
                     for j in range(ND)]
                tot = jnp.sum(_tree_sum(v))
                tot2 = jnp.sum(_tree_sum([u * u for u in v]))
                mean = tot * (1.0 / D)
                var = tot2 * (1.0 / D) - mean * mean + EPS
                rstd = _rsqrt(var)
                for j in range(ND):
                    ag = rstd * g[j]
                    bb = b[j] - mean * ag
                    rows_v[t, pl.ds(j * L, L)] = v[j] * ag + bb
                return carry2

            lax.fori_loop(0, CHUNK, tok_body, 0, unroll=2)
            pltpu.sync_copy(rows_v, out_hbm.at[pl.ds(base + c * CHUNK, CHUNK)])
            return carry

        lax.fori_loop(0, chunks_per_w, chunk_body, 0)

    return sc_kernel


def kernel(x, tok_table, pos_table, gamma, beta):
    bsz, seq = x.shape
    total = bsz * seq
    info = plsc.get_sparse_core_info()
    nw = info.num_cores * info.num_subcores
    x2 = x.reshape(nw, total // (nw * CHUNK), CHUNK)
    pos2 = jnp.concatenate([pos_table[:seq], pos_table[:seq]], axis=0)
    gb = jnp.stack([gamma, beta], axis=0)
    out = _make_sc_kernel(total)(x2, tok_table, pos2, gb)
    return out.reshape(bsz, seq, D)


# speedup vs baseline: 2.1617x; 1.1776x over previous
import functools

import jax
import jax.numpy as jnp
from jax import lax
from jax.experimental import pallas as pl
from jax.experimental.pallas import tpu as pltpu
from jax.experimental.pallas import tpu_sc as plsc

D = 128
L = 16
ND = D // L
S = 200
CHUNK = 128
EPS = 1e-12


def _rsqrt(x):
    i = lax.bitcast_convert_type(x, jnp.int32)
    i = jnp.int32(0x5F3759DF) - lax.shift_right_logical(i, 1)
    y = lax.bitcast_convert_type(i, jnp.float32)
    for _ in range(3):
        y = y * (1.5 - 0.5 * x * y * y)
    return y


def _tree_sum(vs):
    while len(vs) > 1:
        vs = [a + b for a, b in zip(vs[::2], vs[1::2])]
    return vs[0]


def _make_sc_kernel(total_tokens):
    info = plsc.get_sparse_core_info()
    nc, ns = info.num_cores, info.num_subcores
    nw = nc * ns
    tok_per_w = total_tokens // nw
    chunks_per_w = tok_per_w // CHUNK

    mesh = plsc.VectorSubcoreMesh(core_axis_name="c", subcore_axis_name="s")

    @functools.partial(
        pl.kernel,
        mesh=mesh,
        out_type=jax.ShapeDtypeStruct((total_tokens, D), jnp.float32),
        compiler_params=pltpu.CompilerParams(needs_layout_passes=False),
        scratch_types=[
            pltpu.VMEM((chunks_per_w, CHUNK), jnp.int32),
            pltpu.VMEM((2 * S, D), jnp.float32),
            pltpu.VMEM((CHUNK, D), jnp.float32),
            pltpu.VMEM((CHUNK, D), jnp.float32),
            pltpu.VMEM((CHUNK, D), jnp.float32),
            pltpu.VMEM((CHUNK, D), jnp.float32),
            pltpu.VMEM((2, D), jnp.float32),
            pltpu.SemaphoreType.DMA,
            pltpu.SemaphoreType.DMA,
            pltpu.SemaphoreType.DMA,
            pltpu.SemaphoreType.DMA,
        ],
    )
    def sc_kernel(x_hbm, tok_hbm, pos2_hbm, gb_hbm, out_hbm,
                  idx_v, pos_v, g0_v, g1_v, o0_v, o1_v, gb_v,
                  sg0, sg1, so0, so1):
        wid = lax.axis_index("s") * nc + lax.axis_index("c")
        base = wid * tok_per_w

        pltpu.sync_copy(x_hbm.at[wid], idx_v)
        pltpu.sync_copy(pos2_hbm, pos_v)
        pltpu.sync_copy(gb_hbm, gb_v)
        g = [gb_v[0, pl.ds(j * L, L)] for j in range(ND)]
        b = [gb_v[1, pl.ds(j * L, L)] for j in range(ND)]

        def fire_g(c, buf, sem):
            pltpu.async_copy(tok_hbm.at[idx_v.at[c]], buf, sem)

        def wait_g(c, buf, sem):
            pltpu.make_async_copy(tok_hbm.at[idx_v.at[c]], buf, sem).wait()

        def fire_o(c, buf, sem):
            pltpu.async_copy(buf, out_hbm.at[pl.ds(base + c * CHUNK, CHUNK)], sem)

        def wait_o(c, buf, sem):
            pltpu.make_async_copy(
                buf, out_hbm.at[pl.ds(base + c * CHUNK, CHUNK)], sem).wait()

        def compute(c, gbuf, obuf):
            s0 = lax.rem(c * CHUNK, S)

            def tok_body(t, carry2):
                p = s0 + t
                v = [gbuf[t, pl.ds(j * L, L)] + pos_v[p, pl.ds(j * L, L)]
                     for j in range(ND)]
                tot = jnp.sum(_tree_sum(v))
                tot2 = jnp.sum(_tree_sum([u * u for u in v]))
                mean = tot * (1.0 / D)
                var = tot2 * (1.0 / D) - mean * mean + EPS
                rstd = _rsqrt(var)
                for j in range(ND):
                    ag = rstd * g[j]
                    bb = b[j] - mean * ag
                    obuf[t, pl.ds(j * L, L)] = v[j] * ag + bb
                return carry2

            lax.fori_loop(0, CHUNK, tok_body, 0, unroll=2)

        fire_g(0, g0_v, sg0)
        fire_g(1, g1_v, sg1)

        def pair_body(p, carry):
            for k, (gbuf, obuf, sg, so) in enumerate(
                    ((g0_v, o0_v, sg0, so0), (g1_v, o1_v, sg1, so1))):
                c = 2 * p + k
                wait_g(c, gbuf, sg)

                @pl.when(c >= 2)
                def _():
                    wait_o(c - 2, obuf, so)

                compute(c, gbuf, obuf)
                fire_o(c, obuf, so)

                @pl.when(c + 2 < chunks_per_w)
                def _():
                    fire_g(c + 2, gbuf, sg)
            return carry

        lax.fori_loop(0, chunks_per_w // 2, pair_body, 0)
        wait_o(chunks_per_w - 2, o0_v, so0)
        wait_o(chunks_per_w - 1, o1_v, so1)

    return sc_kernel


def kernel(x, tok_table, pos_table, gamma, beta):
    bsz, seq = x.shape
    total = bsz * seq
    info = plsc.get_sparse_core_info()
    nw = info.num_cores * info.num_subcores
    x2 = x.reshape(nw, total // (nw * CHUNK), CHUNK)
    pos2 = jnp.concatenate([pos_table[:seq], pos_table[:seq]], axis=0)
    gb = jnp.stack([gamma, beta], axis=0)
    out = _make_sc_kernel(total)(x2, tok_table, pos2, gb)
    return out.reshape(bsz, seq, D)


# speedup vs baseline: 2.4187x; 1.1189x over previous
import functools

import jax
import jax.numpy as jnp
from jax import lax
from jax.experimental import pallas as pl
from jax.experimental.pallas import tpu as pltpu
from jax.experimental.pallas import tpu_sc as plsc

D = 128
L = 16
ND = D // L
S = 200
CHUNK = 128
EPS = 1e-12


def _rsqrt_vec(x):
    i = plsc.bitcast(x, jnp.int32)
    i = jnp.int32(0x5F3759DF) - lax.shift_right_logical(i, 1)
    y = plsc.bitcast(i, jnp.float32)
    for _ in range(3):
        y = y * (1.5 - 0.5 * x * y * y)
    return y


_GDN = lax.GatherDimensionNumbers(
    offset_dims=(), collapsed_slice_dims=(0,), start_index_map=(0,))


def _shuffle(v, perm):
    return lax.gather(v, perm[:, None], _GDN, (1,),
                      mode=lax.GatherScatterMode.PROMISE_IN_BOUNDS)


def _splat_sum(v, iota):
    for k in (8, 4, 2, 1):
        v = v + _shuffle(v, iota ^ k)
    return v


def _tree_sum(vs):
    while len(vs) > 1:
        vs = [a + b for a, b in zip(vs[::2], vs[1::2])]
    return vs[0]


def _make_sc_kernel(total_tokens):
    info = plsc.get_sparse_core_info()
    nc, ns = info.num_cores, info.num_subcores
    nw = nc * ns
    tok_per_w = total_tokens // nw
    chunks_per_w = tok_per_w // CHUNK

    mesh = plsc.VectorSubcoreMesh(core_axis_name="c", subcore_axis_name="s")

    @functools.partial(
        pl.kernel,
        mesh=mesh,
        out_type=jax.ShapeDtypeStruct((total_tokens, D), jnp.float32),
        compiler_params=pltpu.CompilerParams(needs_layout_passes=False),
        scratch_types=[
            pltpu.VMEM((chunks_per_w, CHUNK), jnp.int32),
            pltpu.VMEM((2 * S, D), jnp.float32),
            pltpu.VMEM((CHUNK, D), jnp.float32),
            pltpu.VMEM((CHUNK, D), jnp.float32),
            pltpu.VMEM((CHUNK, D), jnp.float32),
            pltpu.VMEM((CHUNK, D), jnp.float32),
            pltpu.VMEM((2, D), jnp.float32),
            pltpu.SemaphoreType.DMA,
            pltpu.SemaphoreType.DMA,
            pltpu.SemaphoreType.DMA,
            pltpu.SemaphoreType.DMA,
        ],
    )
    def sc_kernel(x_hbm, tok_hbm, pos2_hbm, gb_hbm, out_hbm,
                  idx_v, pos_v, g0_v, g1_v, o0_v, o1_v, gb_v,
                  sg0, sg1, so0, so1):
        wid = lax.axis_index("s") * nc + lax.axis_index("c")
        base = wid * tok_per_w

        pltpu.sync_copy(x_hbm.at[wid], idx_v)
        pltpu.sync_copy(pos2_hbm, pos_v)
        pltpu.sync_copy(gb_hbm, gb_v)
        g = [gb_v[0, pl.ds(j * L, L)] for j in range(ND)]
        b = [gb_v[1, pl.ds(j * L, L)] for j in range(ND)]

        def fire_g(c, buf, sem):
            pltpu.async_copy(tok_hbm.at[idx_v.at[c]], buf, sem)

        def wait_g(c, buf, sem):
            pltpu.make_async_copy(tok_hbm.at[idx_v.at[c]], buf, sem).wait()

        def fire_o(c, buf, sem):
            pltpu.async_copy(buf, out_hbm.at[pl.ds(base + c * CHUNK, CHUNK)], sem)

        def wait_o(c, buf, sem):
            pltpu.make_async_copy(
                buf, out_hbm.at[pl.ds(base + c * CHUNK, CHUNK)], sem).wait()

        iota = lax.iota(jnp.int32, L)

        def compute(c, gbuf, obuf):
            s0 = lax.rem(c * CHUNK, S)

            def tok_body(t, carry2):
                p = s0 + t
                v = [gbuf[t, pl.ds(j * L, L)] + pos_v[p, pl.ds(j * L, L)]
                     for j in range(ND)]
                tot = _splat_sum(_tree_sum(v), iota)
                tot2 = _splat_sum(_tree_sum([u * u for u in v]), iota)
                mean = tot * (1.0 / D)
                var = tot2 * (1.0 / D) - mean * mean + EPS
                rstd = _rsqrt_vec(var)
                for j in range(ND):
                    ag = rstd * g[j]
                    bb = b[j] - mean * ag
                    obuf[t, pl.ds(j * L, L)] = v[j] * ag + bb
                return carry2

            lax.fori_loop(0, CHUNK, tok_body, 0, unroll=2)

        fire_g(0, g0_v, sg0)
        fire_g(1, g1_v, sg1)

        def pair_body(p, carry):
            for k, (gbuf, obuf, sg, so) in enumerate(
                    ((g0_v, o0_v, sg0, so0), (g1_v, o1_v, sg1, so1))):
                c = 2 * p + k
                wait_g(c, gbuf, sg)

                @pl.when(c >= 2)
                def _():
                    wait_o(c - 2, obuf, so)

                compute(c, gbuf, obuf)
                fire_o(c, obuf, so)

                @pl.when(c + 2 < chunks_per_w)
                def _():
                    fire_g(c + 2, gbuf, sg)
            return carry

        lax.fori_loop(0, chunks_per_w // 2, pair_body, 0)
        wait_o(chunks_per_w - 2, o0_v, so0)
        wait_o(chunks_per_w - 1, o1_v, so1)

    return sc_kernel


def kernel(x, tok_table, pos_table, gamma, beta):
    bsz, seq = x.shape
    total = bsz * seq
    info = plsc.get_sparse_core_info()
    nw = info.num_cores * info.num_subcores
    x2 = x.reshape(nw, total // (nw * CHUNK), CHUNK)
    pos2 = jnp.concatenate([pos_table[:seq], pos_table[:seq]], axis=0)
    gb = jnp.stack([gamma, beta], axis=0)
    out = _make_sc_kernel(total)(x2, tok_table, pos2, gb)
    return out.reshape(bsz, seq, D)


# speedup vs baseline: 4.3306x; 1.7905x over previous
import functools

import jax
import jax.numpy as jnp
from jax import lax
from jax.experimental import pallas as pl
from jax.experimental.pallas import tpu as pltpu
from jax.experimental.pallas import tpu_sc as plsc

D = 128
L = 16
ND = D // L
S = 200
CHUNK = 128
EPS = 1e-12


def _rsqrt_vec(x):
    i = plsc.bitcast(x, jnp.int32)
    i = jnp.int32(0x5F3759DF) - lax.shift_right_logical(i, 1)
    y = plsc.bitcast(i, jnp.float32)
    for _ in range(3):
        y = y * (1.5 - 0.5 * x * y * y)
    return y


_GDN = lax.GatherDimensionNumbers(
    offset_dims=(), collapsed_slice_dims=(0,), start_index_map=(0,))


def _shuffle(v, perm):
    return lax.gather(v, perm[:, None], _GDN, (1,),
                      mode=lax.GatherScatterMode.PROMISE_IN_BOUNDS)


def _splat_sum(v, iota):
    for k in (8, 4, 2, 1):
        v = v + _shuffle(v, iota ^ k)
    return v


def _tree_sum(vs):
    while len(vs) > 1:
        vs = [a + b for a, b in zip(vs[::2], vs[1::2])]
    return vs[0]


def _make_sc_kernel(total_tokens):
    info = plsc.get_sparse_core_info()
    nc, ns = info.num_cores, info.num_subcores
    nw = nc * ns
    tok_per_w = total_tokens // nw
    chunks_per_w = tok_per_w // CHUNK

    mesh = plsc.VectorSubcoreMesh(core_axis_name="c", subcore_axis_name="s")

    @functools.partial(
        pl.kernel,
        mesh=mesh,
        out_type=jax.ShapeDtypeStruct((total_tokens, D), jnp.float32),
        compiler_params=pltpu.CompilerParams(needs_layout_passes=False),
        scratch_types=[
            pltpu.VMEM((chunks_per_w, CHUNK), jnp.int32),
            pltpu.VMEM((2 * S, D), jnp.float32),
            pltpu.VMEM((CHUNK, D), jnp.float32),
            pltpu.VMEM((CHUNK, D), jnp.float32),
            pltpu.VMEM((CHUNK, D), jnp.float32),
            pltpu.VMEM((CHUNK, D), jnp.float32),
            pltpu.VMEM((2, D), jnp.float32),
            pltpu.SemaphoreType.DMA,
            pltpu.SemaphoreType.DMA,
            pltpu.SemaphoreType.DMA,
            pltpu.SemaphoreType.DMA,
        ],
    )
    def sc_kernel(x_hbm, tok_hbm, pos2_hbm, gb_hbm, out_hbm,
                  idx_v, pos_v, g0_v, g1_v, o0_v, o1_v, gb_v,
                  sg0, sg1, so0, so1):
        wid = lax.axis_index("s") * nc + lax.axis_index("c")
        base = wid * tok_per_w

        pltpu.sync_copy(x_hbm.at[wid], idx_v)
        pltpu.sync_copy(pos2_hbm, pos_v)
        pltpu.sync_copy(gb_hbm, gb_v)
        g = [gb_v[0, pl.ds(j * L, L)] for j in range(ND)]
        b = [gb_v[1, pl.ds(j * L, L)] for j in range(ND)]

        def fire_g(c, buf, sem):
            pltpu.async_copy(tok_hbm.at[idx_v.at[c]], buf, sem)

        def wait_g(c, buf, sem):
            pltpu.make_async_copy(tok_hbm.at[idx_v.at[c]], buf, sem).wait()

        def fire_o(c, buf, sem):
            pltpu.async_copy(buf, out_hbm.at[pl.ds(base + c * CHUNK, CHUNK)], sem)

        def wait_o(c, buf, sem):
            pltpu.make_async_copy(
                buf, out_hbm.at[pl.ds(base + c * CHUNK, CHUNK)], sem).wait()

        iota = lax.iota(jnp.int32, L)

        def compute(c, gbuf, obuf):
            s0 = lax.rem(c * CHUNK, S)

            @plsc.parallel_loop(0, CHUNK, 1, unroll=4)
            def tok_body(t):
                p = s0 + t
                v = [gbuf[t, pl.ds(j * L, L)] + pos_v[p, pl.ds(j * L, L)]
                     for j in range(ND)]
                tot = _splat_sum(_tree_sum(v), iota)
                tot2 = _splat_sum(_tree_sum([u * u for u in v]), iota)
                mean = tot * (1.0 / D)
                var = tot2 * (1.0 / D) - mean * mean + EPS
                rstd = _rsqrt_vec(var)
                for j in range(ND):
                    ag = rstd * g[j]
                    bb = b[j] - mean * ag
                    obuf[t, pl.ds(j * L, L)] = v[j] * ag + bb

        fire_g(0, g0_v, sg0)
        fire_g(1, g1_v, sg1)

        def pair_body(p, carry):
            for k, (gbuf, obuf, sg, so) in enumerate(
                    ((g0_v, o0_v, sg0, so0), (g1_v, o1_v, sg1, so1))):
                c = 2 * p + k
                wait_g(c, gbuf, sg)

                @pl.when(c >= 2)
                def _():
                    wait_o(c - 2, obuf, so)

                compute(c, gbuf, obuf)
                fire_o(c, obuf, so)

                @pl.when(c + 2 < chunks_per_w)
                def _():
                    fire_g(c + 2, gbuf, sg)
            return carry

        lax.fori_loop(0, chunks_per_w // 2, pair_body, 0)
        wait_o(chunks_per_w - 2, o0_v, so0)
        wait_o(chunks_per_w - 1, o1_v, so1)

    return sc_kernel


def kernel(x, tok_table, pos_table, gamma, beta):
    bsz, seq = x.shape
    total = bsz * seq
    info = plsc.get_sparse_core_info()
    nw = info.num_cores * info.num_subcores
    x2 = x.reshape(nw, total // (nw * CHUNK), CHUNK)
    pos2 = jnp.concatenate([pos_table[:seq], pos_table[:seq]], axis=0)
    gb = jnp.stack([gamma, beta], axis=0)
    out = _make_sc_kernel(total)(x2, tok_table, pos2, gb)
    return out.reshape(bsz, seq, D)
